# Initial kernel scaffold; baseline (speedup 1.0000x reference)
import jax
import jax.numpy as jnp
from jax.experimental import pallas as pl


def kernel(question_seq, correctness_seq, q_matrix, E_q, E_c, h0, W1, b1, W2, b2, W3, b3, W4, b4, Wab, bab, Wdiff, bdiff, Wdisc, bdisc):
    raise NotImplementedError("write your pallas kernel here")


# speedup vs baseline: 1.3428x; 1.3428x over previous
import jax
import jax.numpy as jnp
from jax.experimental import pallas as pl
from jax.experimental.pallas import tpu as pltpu

_B, _S = 64, 128
_NQ, _C = 2000, 128
_K, _DE, _DC = 64, 64, 64
_BBLK = 32
_NPROG = _B // _BBLK


def _lpkt_body(qseq_ref, cseq_ref, table_ref, Ec_ref, h0_ref,
               W1a_ref, W1b_ref, b1_ref,
               W2a_ref, W2b_ref, W2c_ref, b2_ref,
               W3a_ref, W3b_ref, W3c_ref, b3_ref,
               W4a_ref, W4b_ref, b4_ref,
               wab_ref, bab_ref, Wdiff_ref, bdiff_ref,
               wdisc_ref, bdisc_ref,
               out_ref,
               q_all, learn_all, diff_all, disc_all, h_scr):
    f32 = jnp.float32
    iota_nq = jax.lax.broadcasted_iota(jnp.int32, (_NQ, _BBLK), 0)
    iota_2 = jax.lax.broadcasted_iota(jnp.int32, (2, _BBLK), 0)
    lane_iota = jax.lax.broadcasted_iota(jnp.int32, (1, _S), 1)

    def pre_body(t, _):
        idx = qseq_ref[pl.ds(0, 1), pl.ds(t, 1), :][0]
        ohT = (iota_nq == idx).astype(f32)
        ge = jax.lax.dot_general(ohT, table_ref[...],
                                 (((0,), (0,)), ((), ())),
                                 preferred_element_type=f32)
        qrow = ge[:, :_C]
        e = ge[:, _C:]
        cidx = cseq_ref[pl.ds(0, 1), pl.ds(t, 1), :][0]
        ohc = (iota_2 == cidx).astype(f32)
        c_emb = jax.lax.dot_general(ohc, Ec_ref[...],
                                    (((0,), (0,)), ((), ())),
                                    preferred_element_type=f32)
        learning = (jnp.dot(e, W1a_ref[...], preferred_element_type=f32)
                    + jnp.dot(c_emb, W1b_ref[...], preferred_element_type=f32)
                    + b1_ref[...])
        diff = jax.nn.sigmoid(
            jnp.dot(e, Wdiff_ref[...], preferred_element_type=f32)
            + bdiff_ref[...])
        dl = jnp.sum(e * wdisc_ref[...], axis=1, keepdims=True) + bdisc_ref[...]
        disc = jax.nn.sigmoid(dl) * 5.0
        q_all[pl.ds(t, 1)] = qrow[None]
        learn_all[pl.ds(t, 1)] = learning[None]
        diff_all[pl.ds(t, 1)] = diff[None]
        disc_all[pl.ds(t, 1)] = jnp.broadcast_to(disc, (_BBLK, _C))[None]
        return 0

    jax.lax.fori_loop(0, _S, pre_body, 0)

    h0 = h0_ref[...]
    h_scr[...] = jnp.broadcast_to(h0[None], (_BBLK, _C, _K))
    q0 = q_all[pl.ds(0, 1)][0]
    h_tilde0 = jnp.dot(q0, h0, preferred_element_type=f32)
    wab3 = wab_ref[...][None]

    def step(t, carry):
        h_tilde_pre, learning_pre, ys = carry
        learning = learn_all[pl.ds(t - 1, 1)][0]
        q_e = q_all[pl.ds(t - 1, 1)][0]
        q_next = q_all[pl.ds(t, 1)][0]
        lg_lin = (jnp.dot(learning_pre, W2a_ref[...], preferred_element_type=f32)
                  + jnp.dot(learning, W2b_ref[...], preferred_element_type=f32)
                  + jnp.dot(h_tilde_pre, W2c_ref[...], preferred_element_type=f32)
                  + b2_ref[...])
        gl_lin = (jnp.dot(learning_pre, W3a_ref[...], preferred_element_type=f32)
                  + jnp.dot(learning, W3b_ref[...], preferred_element_type=f32)
                  + jnp.dot(h_tilde_pre, W3c_ref[...], preferred_element_type=f32)
                  + b3_ref[...])
        LG = jax.nn.sigmoid(gl_lin) * (jnp.tanh(lg_lin) + 1.0) * 0.5
        h_pre = h_scr[...]
        hp2 = h_pre.reshape(_BBLK * _C, _K)
        gf_lin = (jnp.dot(hp2, W4a_ref[...],
                          preferred_element_type=f32).reshape(_BBLK, _C, _K)
                  + (jnp.dot(LG, W4b_ref[...], preferred_element_type=f32)
                     + b4_ref[...])[:, None, :])
        gamma_f = jax.nn.sigmoid(gf_lin)
        h = q_e[:, :, None] * LG[:, None, :] + gamma_f * h_pre
        h_scr[...] = h
        al = jnp.sum(h * wab3, axis=2) + bab_ref[...]
        ability = jax.nn.sigmoid(al) * q_next
        diff = diff_all[pl.ds(t, 1)][0] * q_next
        disc = disc_all[pl.ds(t, 1)][0]
        yl = jnp.sum(disc * (ability - diff), axis=1, keepdims=True)
        y = jax.nn.sigmoid(yl)
        ys = ys + jnp.where(lane_iota == t, y, 0.0)
        h_tilde = jnp.sum(q_next[:, :, None] * h, axis=1)
        return (h_tilde, learning, ys)

    init = (h_tilde0, jnp.zeros((_BBLK, _K), f32), jnp.zeros((_BBLK, _S), f32))
    _, _, ys = jax.lax.fori_loop(1, _S, step, init)
    out_ref[...] = ys


def kernel(question_seq, correctness_seq, q_matrix, E_q, E_c, h0,
           W1, b1, W2, b2, W3, b3, W4, b4,
           Wab, bab, Wdiff, bdiff, Wdisc, bdisc):
    f32 = jnp.float32
    qseq_r = question_seq.T.reshape(_S, _NPROG, _BBLK).transpose(1, 0, 2)
    cseq_r = correctness_seq.T.reshape(_S, _NPROG, _BBLK).transpose(1, 0, 2)
    table = jnp.concatenate([q_matrix, E_q], axis=1)

    full = lambda shape: pl.BlockSpec(shape, lambda i: (0,) * len(shape))
    grid_spec = pltpu.PrefetchScalarGridSpec(
        num_scalar_prefetch=0,
        grid=(_NPROG,),
        in_specs=[
            pl.BlockSpec((1, _S, _BBLK), lambda i: (i, 0, 0)),
            pl.BlockSpec((1, _S, _BBLK), lambda i: (i, 0, 0)),
            full((_NQ, _C + _DE)),
            full((2, _DC)),
            full((_C, _K)),
            full((_DE, _K)), full((_DC, _K)), full((1, _K)),
            full((_K, _K)), full((_K, _K)), full((_K, _K)), full((1, _K)),
            full((_K, _K)), full((_K, _K)), full((_K, _K)), full((1, _K)),
            full((_K, _K)), full((_K, _K)), full((1, _K)),
            full((1, _K)), full((1, 1)),
            full((_DE, _C)), full((1, _C)),
            full((1, _DE)), full((1, 1)),
        ],
        out_specs=pl.BlockSpec((_BBLK, _S), lambda i: (i, 0)),
        scratch_shapes=[
            pltpu.VMEM((_S, _BBLK, _C), f32),
            pltpu.VMEM((_S, _BBLK, _K), f32),
            pltpu.VMEM((_S, _BBLK, _C), f32),
            pltpu.VMEM((_S, _BBLK, _C), f32),
            pltpu.VMEM((_BBLK, _C, _K), f32),
        ],
    )
    pred = pl.pallas_call(
        _lpkt_body,
        grid_spec=grid_spec,
        out_shape=jax.ShapeDtypeStruct((_B, _S), f32),
        compiler_params=pltpu.CompilerParams(
            dimension_semantics=("parallel",),
            vmem_limit_bytes=48 * 1024 * 1024,
        ),
    )(
        qseq_r, cseq_r, table, E_c, h0,
        W1[:_DE], W1[_DE:], b1.reshape(1, _K),
        W2[:_K], W2[_K:2 * _K], W2[2 * _K:], b2.reshape(1, _K),
        W3[:_K], W3[_K:2 * _K], W3[2 * _K:], b3.reshape(1, _K),
        W4[:_K], W4[_K:], b4.reshape(1, _K),
        Wab.reshape(1, _K), bab.reshape(1, 1),
        Wdiff, bdiff.reshape(1, _C),
        Wdisc.reshape(1, _DE), bdisc.reshape(1, 1),
    )
    return pred


# speedup vs baseline: 2.2070x; 1.6435x over previous
import jax
import jax.numpy as jnp
from jax.experimental import pallas as pl
from jax.experimental.pallas import tpu as pltpu

_B, _S = 64, 128
_NQ, _C = 2000, 128
_K, _DE, _DC = 64, 64, 64
_BBLK = 32
_NPROG = _B // _BBLK


def _lpkt_body(qseq_ref, cseq_ref, table_ref, Ec_ref, h0_ref,
               W1a_ref, W1b_ref, b1_ref,
               W2a_ref, W2b_ref, W2c_ref, b2_ref,
               W3a_ref, W3b_ref, W3c_ref, b3_ref,
               W4a_ref, W4b_ref, b4_ref,
               wab_ref, bab_ref, Wdiff_ref, bdiff_ref,
               wdisc_ref, bdisc_ref,
               out_ref,
               q_all, learn_all, diff_all, disc_all, h_scr, al_all):
    f32 = jnp.float32
    iota_nq = jax.lax.broadcasted_iota(jnp.int32, (_NQ, _BBLK), 0)
    iota_2 = jax.lax.broadcasted_iota(jnp.int32, (2, _BBLK), 0)
    lane_iota = jax.lax.broadcasted_iota(jnp.int32, (1, _S), 1)

    def pre_body(t, _):
        idx = qseq_ref[pl.ds(0, 1), pl.ds(t, 1), :][0]
        ohT = (iota_nq == idx).astype(f32)
        ge = jax.lax.dot_general(ohT, table_ref[...],
                                 (((0,), (0,)), ((), ())),
                                 preferred_element_type=f32)
        qrow = ge[:, :_C]
        e = ge[:, _C:]
        cidx = cseq_ref[pl.ds(0, 1), pl.ds(t, 1), :][0]
        ohc = (iota_2 == cidx).astype(f32)
        c_emb = jax.lax.dot_general(ohc, Ec_ref[...],
                                    (((0,), (0,)), ((), ())),
                                    preferred_element_type=f32)
        learning = (jnp.dot(e, W1a_ref[...], preferred_element_type=f32)
                    + jnp.dot(c_emb, W1b_ref[...], preferred_element_type=f32)
                    + b1_ref[...])
        diff = jax.nn.sigmoid(
            jnp.dot(e, Wdiff_ref[...], preferred_element_type=f32)
            + bdiff_ref[...])
        dl = jnp.sum(e * wdisc_ref[...], axis=1, keepdims=True) + bdisc_ref[...]
        disc = jax.nn.sigmoid(dl) * 5.0
        q_all[pl.ds(t, 1)] = qrow[None]
        learn_all[pl.ds(t, 1)] = learning[None]
        diff_all[pl.ds(t, 1)] = diff[None]
        disc_all[pl.ds(t, 1)] = jnp.broadcast_to(disc, (_BBLK, _C))[None]
        return 0

    jax.lax.fori_loop(0, _S, pre_body, 0)

    h0 = h0_ref[...]
    h_scr[...] = jnp.broadcast_to(h0[None], (_BBLK, _C, _K))
    q0 = q_all[pl.ds(0, 1)][0]
    h_tilde0 = jnp.dot(q0, h0, preferred_element_type=f32)
    wab3 = wab_ref[...][None]
    al_all[pl.ds(0, 1)] = jnp.zeros((1, _BBLK, _C), f32)

    def step(t, carry):
        h_tilde_pre, learning_pre, learning, q_e = carry
        q_next = q_all[pl.ds(t, 1)][0]
        lg_lin = (jnp.dot(learning_pre, W2a_ref[...], preferred_element_type=f32)
                  + jnp.dot(learning, W2b_ref[...], preferred_element_type=f32)
                  + jnp.dot(h_tilde_pre, W2c_ref[...], preferred_element_type=f32)
                  + b2_ref[...])
        gl_lin = (jnp.dot(learning_pre, W3a_ref[...], preferred_element_type=f32)
                  + jnp.dot(learning, W3b_ref[...], preferred_element_type=f32)
                  + jnp.dot(h_tilde_pre, W3c_ref[...], preferred_element_type=f32)
                  + b3_ref[...])
        LG = jax.nn.sigmoid(gl_lin) * (jnp.tanh(lg_lin) + 1.0) * 0.5
        h_pre = h_scr[...]
        hp2 = h_pre.reshape(_BBLK * _C, _K)
        gf_lin = (jnp.dot(hp2, W4a_ref[...],
                          preferred_element_type=f32).reshape(_BBLK, _C, _K)
                  + (jnp.dot(LG, W4b_ref[...], preferred_element_type=f32)
                     + b4_ref[...])[:, None, :])
        h = (q_e[:, :, None] * LG[:, None, :]
             + jax.nn.sigmoid(gf_lin) * h_pre)
        h_scr[...] = h
        al_all[pl.ds(t, 1)] = jnp.sum(h * wab3, axis=2)[None]
        h_tilde = jnp.sum(q_next[:, :, None] * h, axis=1)
        learning_next = learn_all[pl.ds(t, 1)][0]
        return (h_tilde, learning, learning_next, q_next)

    init = (h_tilde0, jnp.zeros((_BBLK, _K), f32),
            learn_all[pl.ds(0, 1)][0], q0)
    jax.lax.fori_loop(1, _S, step, init)

    ab = jax.nn.sigmoid(al_all[...] + bab_ref[...])
    q3 = q_all[...]
    contrib = disc_all[...] * (ab - diff_all[...]) * q3
    yl = jnp.sum(contrib, axis=2)
    y = jax.nn.sigmoid(yl)
    out_ref[...] = jnp.where(lane_iota == 0, 0.0, y.T)


def kernel(question_seq, correctness_seq, q_matrix, E_q, E_c, h0,
           W1, b1, W2, b2, W3, b3, W4, b4,
           Wab, bab, Wdiff, bdiff, Wdisc, bdisc):
    f32 = jnp.float32
    qseq_r = question_seq.T.reshape(_S, _NPROG, _BBLK).transpose(1, 0, 2)
    cseq_r = correctness_seq.T.reshape(_S, _NPROG, _BBLK).transpose(1, 0, 2)
    table = jnp.concatenate([q_matrix, E_q], axis=1)

    full = lambda shape: pl.BlockSpec(shape, lambda i: (0,) * len(shape))
    grid_spec = pltpu.PrefetchScalarGridSpec(
        num_scalar_prefetch=0,
        grid=(_NPROG,),
        in_specs=[
            pl.BlockSpec((1, _S, _BBLK), lambda i: (i, 0, 0)),
            pl.BlockSpec((1, _S, _BBLK), lambda i: (i, 0, 0)),
            full((_NQ, _C + _DE)),
            full((2, _DC)),
            full((_C, _K)),
            full((_DE, _K)), full((_DC, _K)), full((1, _K)),
            full((_K, _K)), full((_K, _K)), full((_K, _K)), full((1, _K)),
            full((_K, _K)), full((_K, _K)), full((_K, _K)), full((1, _K)),
            full((_K, _K)), full((_K, _K)), full((1, _K)),
            full((1, _K)), full((1, 1)),
            full((_DE, _C)), full((1, _C)),
            full((1, _DE)), full((1, 1)),
        ],
        out_specs=pl.BlockSpec((_BBLK, _S), lambda i: (i, 0)),
        scratch_shapes=[
            pltpu.VMEM((_S, _BBLK, _C), f32),
            pltpu.VMEM((_S, _BBLK, _K), f32),
            pltpu.VMEM((_S, _BBLK, _C), f32),
            pltpu.VMEM((_S, _BBLK, _C), f32),
            pltpu.VMEM((_BBLK, _C, _K), f32),
            pltpu.VMEM((_S, _BBLK, _C), f32),
        ],
    )
    pred = pl.pallas_call(
        _lpkt_body,
        grid_spec=grid_spec,
        out_shape=jax.ShapeDtypeStruct((_B, _S), f32),
        compiler_params=pltpu.CompilerParams(
            dimension_semantics=("parallel",),
            vmem_limit_bytes=48 * 1024 * 1024,
        ),
    )(
        qseq_r, cseq_r, table, E_c, h0,
        W1[:_DE], W1[_DE:], b1.reshape(1, _K),
        W2[:_K], W2[_K:2 * _K], W2[2 * _K:], b2.reshape(1, _K),
        W3[:_K], W3[_K:2 * _K], W3[2 * _K:], b3.reshape(1, _K),
        W4[:_K], W4[_K:], b4.reshape(1, _K),
        Wab.reshape(1, _K), bab.reshape(1, 1),
        Wdiff, bdiff.reshape(1, _C),
        Wdisc.reshape(1, _DE), bdisc.reshape(1, 1),
    )
    return pred


# speedup vs baseline: 2.7026x; 1.2246x over previous
import jax
import jax.numpy as jnp
from jax.experimental import pallas as pl
from jax.experimental.pallas import tpu as pltpu

_B, _S = 64, 128
_NQ, _C = 2000, 128
_K, _DE, _DC = 64, 64, 64
_BBLK = 32
_NPROG = _B // _BBLK


def _lpkt_body(qseq_ref, cseq_ref, table_ref, Ec_ref, h0_ref,
               W1a_ref, W1b_ref, b1_ref,
               W2a_ref, W2b_ref, W2c_ref, b2_ref,
               W3a_ref, W3b_ref, W3c_ref, b3_ref,
               W4a_ref, W4b_ref, b4_ref,
               wab_ref, bab_ref, Wdiff_ref, bdiff_ref,
               wdisc_ref, bdisc_ref,
               out_ref,
               q_all, learn_all, diff_all, disc_all, h_scr, al_all):
    f32 = jnp.float32
    iota_nq = jax.lax.broadcasted_iota(jnp.int32, (_NQ, _BBLK), 0)
    iota_2 = jax.lax.broadcasted_iota(jnp.int32, (2, _BBLK), 0)
    lane_iota = jax.lax.broadcasted_iota(jnp.int32, (1, _S), 1)

    def pre_body(t, _):
        idx = qseq_ref[pl.ds(0, 1), pl.ds(t, 1), :][0]
        ohT = (iota_nq == idx).astype(f32)
        ge = jax.lax.dot_general(ohT, table_ref[...],
                                 (((0,), (0,)), ((), ())),
                                 preferred_element_type=f32)
        qrow = ge[:, :_C]
        e = ge[:, _C:]
        cidx = cseq_ref[pl.ds(0, 1), pl.ds(t, 1), :][0]
        ohc = (iota_2 == cidx).astype(f32)
        c_emb = jax.lax.dot_general(ohc, Ec_ref[...],
                                    (((0,), (0,)), ((), ())),
                                    preferred_element_type=f32)
        learning = (jnp.dot(e, W1a_ref[...], preferred_element_type=f32)
                    + jnp.dot(c_emb, W1b_ref[...], preferred_element_type=f32)
                    + b1_ref[...])
        diff = jax.nn.sigmoid(
            jnp.dot(e, Wdiff_ref[...], preferred_element_type=f32)
            + bdiff_ref[...])
        dl = jnp.sum(e * wdisc_ref[...], axis=1, keepdims=True) + bdisc_ref[...]
        disc = jax.nn.sigmoid(dl) * 5.0
        q_all[pl.ds(t, 1)] = qrow[None]
        learn_all[pl.ds(t, 1)] = learning[None]
        diff_all[pl.ds(t, 1)] = diff[None]
        disc_all[pl.ds(t, 1)] = jnp.broadcast_to(disc, (_BBLK, _C))[None]
        return 0

    jax.lax.fori_loop(0, _S, pre_body, 0)

    h0 = h0_ref[...]
    h_scr[...] = jnp.broadcast_to(h0[None], (_BBLK, _C, _K))
    q0 = q_all[pl.ds(0, 1)][0]
    h_tilde0 = jnp.dot(q0, h0, preferred_element_type=f32)
    wab3 = wab_ref[...][None]
    al_all[pl.ds(0, 1)] = jnp.zeros((1, _BBLK, _C), f32)
    bdiag = (jax.lax.broadcasted_iota(jnp.int32, (_BBLK, _BBLK * _C), 1)
             // _C == jax.lax.broadcasted_iota(
                 jnp.int32, (_BBLK, _BBLK * _C), 0)).astype(f32)

    def step(t, carry):
        h_tilde_pre, learning_pre, learning, q_e = carry
        q_next = q_all[pl.ds(t, 1)][0]
        lg_lin = (jnp.dot(learning_pre, W2a_ref[...], preferred_element_type=f32)
                  + jnp.dot(learning, W2b_ref[...], preferred_element_type=f32)
                  + jnp.dot(h_tilde_pre, W2c_ref[...], preferred_element_type=f32)
                  + b2_ref[...])
        gl_lin = (jnp.dot(learning_pre, W3a_ref[...], preferred_element_type=f32)
                  + jnp.dot(learning, W3b_ref[...], preferred_element_type=f32)
                  + jnp.dot(h_tilde_pre, W3c_ref[...], preferred_element_type=f32)
                  + b3_ref[...])
        LG = jax.nn.sigmoid(gl_lin) * (jnp.tanh(lg_lin) + 1.0) * 0.5
        h_pre = h_scr[...]
        hp2 = h_pre.reshape(_BBLK * _C, _K)
        gf_lin = (jnp.dot(hp2, W4a_ref[...],
                          preferred_element_type=f32).reshape(_BBLK, _C, _K)
                  + (jnp.dot(LG, W4b_ref[...], preferred_element_type=f32)
                     + b4_ref[...])[:, None, :])
        h = (q_e[:, :, None] * LG[:, None, :]
             + jax.nn.sigmoid(gf_lin) * h_pre)
        h_scr[...] = h
        al_all[pl.ds(t, 1)] = jnp.sum(h * wab3, axis=2)[None]
        sel = bdiag * jnp.tile(q_next, (1, _BBLK))
        h_tilde = jnp.dot(sel, h.reshape(_BBLK * _C, _K),
                          preferred_element_type=f32)
        learning_next = learn_all[pl.ds(t, 1)][0]
        return (h_tilde, learning, learning_next, q_next)

    init = (h_tilde0, jnp.zeros((_BBLK, _K), f32),
            learn_all[pl.ds(0, 1)][0], q0)
    jax.lax.fori_loop(1, _S, step, init)

    ab = jax.nn.sigmoid(al_all[...] + bab_ref[...])
    q3 = q_all[...]
    contrib = disc_all[...] * (ab - diff_all[...]) * q3
    yl = jnp.sum(contrib, axis=2)
    y = jax.nn.sigmoid(yl)
    out_ref[...] = jnp.where(lane_iota == 0, 0.0, y.T)


def kernel(question_seq, correctness_seq, q_matrix, E_q, E_c, h0,
           W1, b1, W2, b2, W3, b3, W4, b4,
           Wab, bab, Wdiff, bdiff, Wdisc, bdisc):
    f32 = jnp.float32
    qseq_r = question_seq.T.reshape(_S, _NPROG, _BBLK).transpose(1, 0, 2)
    cseq_r = correctness_seq.T.reshape(_S, _NPROG, _BBLK).transpose(1, 0, 2)
    table = jnp.concatenate([q_matrix, E_q], axis=1)

    full = lambda shape: pl.BlockSpec(shape, lambda i: (0,) * len(shape))
    grid_spec = pltpu.PrefetchScalarGridSpec(
        num_scalar_prefetch=0,
        grid=(_NPROG,),
        in_specs=[
            pl.BlockSpec((1, _S, _BBLK), lambda i: (i, 0, 0)),
            pl.BlockSpec((1, _S, _BBLK), lambda i: (i, 0, 0)),
            full((_NQ, _C + _DE)),
            full((2, _DC)),
            full((_C, _K)),
            full((_DE, _K)), full((_DC, _K)), full((1, _K)),
            full((_K, _K)), full((_K, _K)), full((_K, _K)), full((1, _K)),
            full((_K, _K)), full((_K, _K)), full((_K, _K)), full((1, _K)),
            full((_K, _K)), full((_K, _K)), full((1, _K)),
            full((1, _K)), full((1, 1)),
            full((_DE, _C)), full((1, _C)),
            full((1, _DE)), full((1, 1)),
        ],
        out_specs=pl.BlockSpec((_BBLK, _S), lambda i: (i, 0)),
        scratch_shapes=[
            pltpu.VMEM((_S, _BBLK, _C), f32),
            pltpu.VMEM((_S, _BBLK, _K), f32),
            pltpu.VMEM((_S, _BBLK, _C), f32),
            pltpu.VMEM((_S, _BBLK, _C), f32),
            pltpu.VMEM((_BBLK, _C, _K), f32),
            pltpu.VMEM((_S, _BBLK, _C), f32),
        ],
    )
    pred = pl.pallas_call(
        _lpkt_body,
        grid_spec=grid_spec,
        out_shape=jax.ShapeDtypeStruct((_B, _S), f32),
        compiler_params=pltpu.CompilerParams(
            dimension_semantics=("parallel",),
            vmem_limit_bytes=48 * 1024 * 1024,
        ),
    )(
        qseq_r, cseq_r, table, E_c, h0,
        W1[:_DE], W1[_DE:], b1.reshape(1, _K),
        W2[:_K], W2[_K:2 * _K], W2[2 * _K:], b2.reshape(1, _K),
        W3[:_K], W3[_K:2 * _K], W3[2 * _K:], b3.reshape(1, _K),
        W4[:_K], W4[_K:], b4.reshape(1, _K),
        Wab.reshape(1, _K), bab.reshape(1, 1),
        Wdiff, bdiff.reshape(1, _C),
        Wdisc.reshape(1, _DE), bdisc.reshape(1, 1),
    )
    return pred


# speedup vs baseline: 3.2549x; 1.2043x over previous
import jax
import jax.numpy as jnp
from jax.experimental import pallas as pl
from jax.experimental.pallas import tpu as pltpu

_B, _S = 64, 128
_NQ, _C = 2000, 128
_K, _DE, _DC = 64, 64, 64
_BBLK = 32
_NPROG = _B // _BBLK


def _lpkt_body(qseq_ref, cseq_ref, table_ref, Ec_ref, h0_ref,
               W1a_ref, W1b_ref, b1_ref,
               W2a_ref, W2b_ref, W2c_ref, b2_ref,
               W3a_ref, W3b_ref, W3c_ref, b3_ref,
               W4a_ref, W4b_ref, b4_ref,
               wab_ref, bab_ref, Wdiff_ref, bdiff_ref,
               wdisc_ref, bdisc_ref,
               out_ref,
               q_all, learn_all, diff_all, disc_all, h_scr, al_all):
    f32 = jnp.float32
    _TC = 4
    _R = _BBLK * _TC
    iota_nq = jax.lax.broadcasted_iota(jnp.int32, (_NQ, _R), 0)
    iota_2 = jax.lax.broadcasted_iota(jnp.int32, (2, _R), 0)
    lane_iota = jax.lax.broadcasted_iota(jnp.int32, (1, _S), 1)

    def pre_body(r, _):
        idx = qseq_ref[pl.ds(0, 1), pl.ds(r, 1), :][0]
        ohT = (iota_nq == idx).astype(f32)
        ge = jax.lax.dot_general(ohT, table_ref[...],
                                 (((0,), (0,)), ((), ())),
                                 preferred_element_type=f32)
        q4 = ge[:, :_C]
        e4 = ge[:, _C:]
        cidx = cseq_ref[pl.ds(0, 1), pl.ds(r, 1), :][0]
        ohc = (iota_2 == cidx).astype(f32)
        c4 = jax.lax.dot_general(ohc, Ec_ref[...],
                                 (((0,), (0,)), ((), ())),
                                 preferred_element_type=f32)
        learning4 = (jnp.dot(e4, W1a_ref[...], preferred_element_type=f32)
                     + jnp.dot(c4, W1b_ref[...], preferred_element_type=f32)
                     + b1_ref[...])
        diff4 = jax.nn.sigmoid(
            jnp.dot(e4, Wdiff_ref[...], preferred_element_type=f32)
            + bdiff_ref[...])
        disc4 = jax.nn.sigmoid(
            jnp.dot(e4, wdisc_ref[...], preferred_element_type=f32)
            + bdisc_ref[...]) * 5.0
        t0 = r * _TC
        q_all[pl.ds(t0, _TC)] = q4.reshape(_TC, _BBLK, _C)
        learn_all[pl.ds(t0, _TC)] = learning4.reshape(_TC, _BBLK, _K)
        diff_all[pl.ds(t0, _TC)] = diff4.reshape(_TC, _BBLK, _C)
        disc_all[pl.ds(t0, _TC)] = disc4.reshape(_TC, _BBLK, _C)
        return 0

    jax.lax.fori_loop(0, _S // _TC, pre_body, 0)

    h0 = h0_ref[...]
    h_scr[...] = jnp.broadcast_to(h0[None], (_BBLK, _C, _K))
    q0 = q_all[pl.ds(0, 1)][0]
    h_tilde0 = jnp.dot(q0, h0, preferred_element_type=f32)
    wab3 = wab_ref[...][None]
    al_all[pl.ds(0, 1)] = jnp.zeros((1, _BBLK, _C), f32)
    bdiag = (jax.lax.broadcasted_iota(jnp.int32, (_BBLK, _BBLK * _C), 1)
             // _C == jax.lax.broadcasted_iota(
                 jnp.int32, (_BBLK, _BBLK * _C), 0)).astype(f32)

    def step(t, carry):
        h_tilde_pre, learning_pre, learning, q_e = carry
        q_next = q_all[pl.ds(t, 1)][0]
        lg_lin = (jnp.dot(learning_pre, W2a_ref[...], preferred_element_type=f32)
                  + jnp.dot(learning, W2b_ref[...], preferred_element_type=f32)
                  + jnp.dot(h_tilde_pre, W2c_ref[...], preferred_element_type=f32)
                  + b2_ref[...])
        gl_lin = (jnp.dot(learning_pre, W3a_ref[...], preferred_element_type=f32)
                  + jnp.dot(learning, W3b_ref[...], preferred_element_type=f32)
                  + jnp.dot(h_tilde_pre, W3c_ref[...], preferred_element_type=f32)
                  + b3_ref[...])
        LG = jax.nn.sigmoid(gl_lin) * (jnp.tanh(lg_lin) + 1.0) * 0.5
        h_pre = h_scr[...]
        hp2 = h_pre.reshape(_BBLK * _C, _K)
        gf_lin = (jnp.dot(hp2, W4a_ref[...],
                          preferred_element_type=f32).reshape(_BBLK, _C, _K)
                  + (jnp.dot(LG, W4b_ref[...], preferred_element_type=f32)
                     + b4_ref[...])[:, None, :])
        h = (q_e[:, :, None] * LG[:, None, :]
             + jax.nn.sigmoid(gf_lin) * h_pre)
        h_scr[...] = h
        al_all[pl.ds(t, 1)] = jnp.sum(h * wab3, axis=2)[None]
        sel = bdiag * jnp.tile(q_next, (1, _BBLK))
        h_tilde = jnp.dot(sel, h.reshape(_BBLK * _C, _K),
                          preferred_element_type=f32)
        learning_next = learn_all[pl.ds(t, 1)][0]
        return (h_tilde, learning, learning_next, q_next)

    init = (h_tilde0, jnp.zeros((_BBLK, _K), f32),
            learn_all[pl.ds(0, 1)][0], q0)
    jax.lax.fori_loop(1, _S, step, init)

    ab = jax.nn.sigmoid(al_all[...] + bab_ref[...])
    q3 = q_all[...]
    contrib = disc_all[...] * (ab - diff_all[...]) * q3
    yl = jnp.sum(contrib, axis=2)
    y = jax.nn.sigmoid(yl)
    out_ref[...] = jnp.where(lane_iota == 0, 0.0, y.T)


def kernel(question_seq, correctness_seq, q_matrix, E_q, E_c, h0,
           W1, b1, W2, b2, W3, b3, W4, b4,
           Wab, bab, Wdiff, bdiff, Wdisc, bdisc):
    f32 = jnp.float32
    qseq_r = (question_seq.T.reshape(_S // 4, 4, _NPROG, _BBLK)
              .transpose(2, 0, 1, 3).reshape(_NPROG, _S // 4, 4 * _BBLK))
    cseq_r = (correctness_seq.T.reshape(_S // 4, 4, _NPROG, _BBLK)
              .transpose(2, 0, 1, 3).reshape(_NPROG, _S // 4, 4 * _BBLK))
    table = jnp.concatenate([q_matrix, E_q], axis=1)

    full = lambda shape: pl.BlockSpec(shape, lambda i: (0,) * len(shape))
    grid_spec = pltpu.PrefetchScalarGridSpec(
        num_scalar_prefetch=0,
        grid=(_NPROG,),
        in_specs=[
            pl.BlockSpec((1, _S // 4, 4 * _BBLK), lambda i: (i, 0, 0)),
            pl.BlockSpec((1, _S // 4, 4 * _BBLK), lambda i: (i, 0, 0)),
            full((_NQ, _C + _DE)),
            full((2, _DC)),
            full((_C, _K)),
            full((_DE, _K)), full((_DC, _K)), full((1, _K)),
            full((_K, _K)), full((_K, _K)), full((_K, _K)), full((1, _K)),
            full((_K, _K)), full((_K, _K)), full((_K, _K)), full((1, _K)),
            full((_K, _K)), full((_K, _K)), full((1, _K)),
            full((1, _K)), full((1, 1)),
            full((_DE, _C)), full((1, _C)),
            full((_DE, _C)), full((1, 1)),
        ],
        out_specs=pl.BlockSpec((_BBLK, _S), lambda i: (i, 0)),
        scratch_shapes=[
            pltpu.VMEM((_S, _BBLK, _C), f32),
            pltpu.VMEM((_S, _BBLK, _K), f32),
            pltpu.VMEM((_S, _BBLK, _C), f32),
            pltpu.VMEM((_S, _BBLK, _C), f32),
            pltpu.VMEM((_BBLK, _C, _K), f32),
            pltpu.VMEM((_S, _BBLK, _C), f32),
        ],
    )
    pred = pl.pallas_call(
        _lpkt_body,
        grid_spec=grid_spec,
        out_shape=jax.ShapeDtypeStruct((_B, _S), f32),
        compiler_params=pltpu.CompilerParams(
            dimension_semantics=("parallel",),
            vmem_limit_bytes=48 * 1024 * 1024,
        ),
    )(
        qseq_r, cseq_r, table, E_c, h0,
        W1[:_DE], W1[_DE:], b1.reshape(1, _K),
        W2[:_K], W2[_K:2 * _K], W2[2 * _K:], b2.reshape(1, _K),
        W3[:_K], W3[_K:2 * _K], W3[2 * _K:], b3.reshape(1, _K),
        W4[:_K], W4[_K:], b4.reshape(1, _K),
        Wab.reshape(1, _K), bab.reshape(1, 1),
        Wdiff, bdiff.reshape(1, _C),
        jnp.tile(Wdisc, (1, _C)), bdisc.reshape(1, 1),
    )
    return pred


# speedup vs baseline: 3.7272x; 1.1451x over previous
import jax
import jax.numpy as jnp
from jax.experimental import pallas as pl
from jax.experimental.pallas import tpu as pltpu

_B, _S = 64, 128
_NQ, _C = 2000, 128
_K, _DE, _DC = 64, 64, 64
_BBLK = 32
_NPROG = _B // _BBLK


def _lpkt_body(qseq_ref, cseq_ref, table_ref, Ec_ref, h0_ref,
               W1a_ref, W1b_ref, b1_ref,
               W2a_ref, W2b_ref, W2c_ref, b2_ref,
               W3a_ref, W3b_ref, W3c_ref, b3_ref,
               W4a_ref, W4b_ref, b4_ref,
               wab_ref, bab_ref, Wdiff_ref, bdiff_ref,
               wdisc_ref, bdisc_ref,
               out_ref,
               q_all, learn_all, diff_all, disc_all, h_scr, al_all):
    f32 = jnp.float32
    _TC = 4
    _R = _BBLK * _TC
    iota_nq = jax.lax.broadcasted_iota(jnp.int32, (_NQ, _R), 0)
    iota_2 = jax.lax.broadcasted_iota(jnp.int32, (2, _R), 0)
    lane_iota = jax.lax.broadcasted_iota(jnp.int32, (1, _S), 1)

    def pre_body(r, _):
        idx = qseq_ref[pl.ds(0, 1), pl.ds(r, 1), :][0]
        ohT = (iota_nq == idx).astype(f32)
        ge = jax.lax.dot_general(ohT, table_ref[...],
                                 (((0,), (0,)), ((), ())),
                                 preferred_element_type=f32)
        q4 = ge[:, :_C]
        e4 = ge[:, _C:]
        cidx = cseq_ref[pl.ds(0, 1), pl.ds(r, 1), :][0]
        ohc = (iota_2 == cidx).astype(f32)
        c4 = jax.lax.dot_general(ohc, Ec_ref[...],
                                 (((0,), (0,)), ((), ())),
                                 preferred_element_type=f32)
        learning4 = (jnp.dot(e4, W1a_ref[...], preferred_element_type=f32)
                     + jnp.dot(c4, W1b_ref[...], preferred_element_type=f32)
                     + b1_ref[...])
        diff4 = jax.nn.sigmoid(
            jnp.dot(e4, Wdiff_ref[...], preferred_element_type=f32)
            + bdiff_ref[...])
        disc4 = jax.nn.sigmoid(
            jnp.dot(e4, wdisc_ref[...], preferred_element_type=f32)
            + bdisc_ref[...]) * 5.0
        t0 = r * _TC
        q_all[pl.ds(t0, _TC)] = q4.reshape(_TC, _BBLK, _C)
        learn_all[pl.ds(t0, _TC)] = learning4.reshape(_TC, _BBLK, _K)
        diff_all[pl.ds(t0, _TC)] = diff4.reshape(_TC, _BBLK, _C)
        disc_all[pl.ds(t0, _TC)] = disc4.reshape(_TC, _BBLK, _C)
        return 0

    jax.lax.fori_loop(0, _S // _TC, pre_body, 0)

    h0 = h0_ref[...]
    h_scr[...] = jnp.broadcast_to(h0[None], (_BBLK, _C, _K))
    q0 = q_all[pl.ds(0, 1)][0]
    h_tilde0 = jnp.dot(q0, h0, preferred_element_type=f32)
    wab3 = wab_ref[...][None]
    al_all[pl.ds(0, 1)] = jnp.zeros((1, _BBLK, _C), f32)
    bdiag = (jax.lax.broadcasted_iota(jnp.int32, (_BBLK, _BBLK * _C), 1)
             // _C == jax.lax.broadcasted_iota(
                 jnp.int32, (_BBLK, _BBLK * _C), 0)).astype(f32)

    def step(t, carry):
        h_tilde_pre, learning_pre, learning, q_e = carry
        q_next = q_all[pl.ds(t, 1)][0]
        lg_lin = (jnp.dot(learning_pre, W2a_ref[...], preferred_element_type=f32)
                  + jnp.dot(learning, W2b_ref[...], preferred_element_type=f32)
                  + jnp.dot(h_tilde_pre, W2c_ref[...], preferred_element_type=f32)
                  + b2_ref[...])
        gl_lin = (jnp.dot(learning_pre, W3a_ref[...], preferred_element_type=f32)
                  + jnp.dot(learning, W3b_ref[...], preferred_element_type=f32)
                  + jnp.dot(h_tilde_pre, W3c_ref[...], preferred_element_type=f32)
                  + b3_ref[...])
        LG = jax.nn.sigmoid(gl_lin) * (jnp.tanh(lg_lin) + 1.0) * 0.5
        h_pre = h_scr[...]
        hp2 = h_pre.reshape(_BBLK * _C, _K)
        gf_lin = (jnp.dot(hp2, W4a_ref[...],
                          preferred_element_type=f32).reshape(_BBLK, _C, _K)
                  + (jnp.dot(LG, W4b_ref[...], preferred_element_type=f32)
                     + b4_ref[...])[:, None, :])
        h = (q_e[:, :, None] * LG[:, None, :]
             + jax.nn.sigmoid(gf_lin) * h_pre)
        h_scr[...] = h
        al_all[pl.ds(t, 1)] = jnp.sum(h * wab3, axis=2)[None]
        sel = bdiag * jnp.tile(q_next, (1, _BBLK))
        h_tilde = jnp.dot(sel, h.reshape(_BBLK * _C, _K),
                          preferred_element_type=f32)
        learning_next = learn_all[pl.ds(t, 1)][0]
        return (h_tilde, learning, learning_next, q_next)

    init = (h_tilde0, jnp.zeros((_BBLK, _K), f32),
            learn_all[pl.ds(0, 1)][0], q0)
    jax.lax.fori_loop(1, _S, step, init, unroll=3)

    ab = jax.nn.sigmoid(al_all[...] + bab_ref[...])
    q3 = q_all[...]
    contrib = disc_all[...] * (ab - diff_all[...]) * q3
    yl = jnp.sum(contrib, axis=2)
    y = jax.nn.sigmoid(yl)
    out_ref[...] = jnp.where(lane_iota == 0, 0.0, y.T)


def kernel(question_seq, correctness_seq, q_matrix, E_q, E_c, h0,
           W1, b1, W2, b2, W3, b3, W4, b4,
           Wab, bab, Wdiff, bdiff, Wdisc, bdisc):
    f32 = jnp.float32
    qseq_r = (question_seq.T.reshape(_S // 4, 4, _NPROG, _BBLK)
              .transpose(2, 0, 1, 3).reshape(_NPROG, _S // 4, 4 * _BBLK))
    cseq_r = (correctness_seq.T.reshape(_S // 4, 4, _NPROG, _BBLK)
              .transpose(2, 0, 1, 3).reshape(_NPROG, _S // 4, 4 * _BBLK))
    table = jnp.concatenate([q_matrix, E_q], axis=1)

    full = lambda shape: pl.BlockSpec(shape, lambda i: (0,) * len(shape))
    grid_spec = pltpu.PrefetchScalarGridSpec(
        num_scalar_prefetch=0,
        grid=(_NPROG,),
        in_specs=[
            pl.BlockSpec((1, _S // 4, 4 * _BBLK), lambda i: (i, 0, 0)),
            pl.BlockSpec((1, _S // 4, 4 * _BBLK), lambda i: (i, 0, 0)),
            full((_NQ, _C + _DE)),
            full((2, _DC)),
            full((_C, _K)),
            full((_DE, _K)), full((_DC, _K)), full((1, _K)),
            full((_K, _K)), full((_K, _K)), full((_K, _K)), full((1, _K)),
            full((_K, _K)), full((_K, _K)), full((_K, _K)), full((1, _K)),
            full((_K, _K)), full((_K, _K)), full((1, _K)),
            full((1, _K)), full((1, 1)),
            full((_DE, _C)), full((1, _C)),
            full((_DE, _C)), full((1, 1)),
        ],
        out_specs=pl.BlockSpec((_BBLK, _S), lambda i: (i, 0)),
        scratch_shapes=[
            pltpu.VMEM((_S, _BBLK, _C), f32),
            pltpu.VMEM((_S, _BBLK, _K), f32),
            pltpu.VMEM((_S, _BBLK, _C), f32),
            pltpu.VMEM((_S, _BBLK, _C), f32),
            pltpu.VMEM((_BBLK, _C, _K), f32),
            pltpu.VMEM((_S, _BBLK, _C), f32),
        ],
    )
    pred = pl.pallas_call(
        _lpkt_body,
        grid_spec=grid_spec,
        out_shape=jax.ShapeDtypeStruct((_B, _S), f32),
        compiler_params=pltpu.CompilerParams(
            dimension_semantics=("parallel",),
            vmem_limit_bytes=48 * 1024 * 1024,
        ),
    )(
        qseq_r, cseq_r, table, E_c, h0,
        W1[:_DE], W1[_DE:], b1.reshape(1, _K),
        W2[:_K], W2[_K:2 * _K], W2[2 * _K:], b2.reshape(1, _K),
        W3[:_K], W3[_K:2 * _K], W3[2 * _K:], b3.reshape(1, _K),
        W4[:_K], W4[_K:], b4.reshape(1, _K),
        Wab.reshape(1, _K), bab.reshape(1, 1),
        Wdiff, bdiff.reshape(1, _C),
        jnp.tile(Wdisc, (1, _C)), bdisc.reshape(1, 1),
    )
    return pred


# speedup vs baseline: 3.8383x; 1.0298x over previous
import jax
import jax.numpy as jnp
from jax.experimental import pallas as pl
from jax.experimental.pallas import tpu as pltpu

_B, _S = 64, 128
_NQ, _C = 2000, 128
_K, _DE, _DC = 64, 64, 64
_BBLK = 32
_NPROG = _B // _BBLK


def _lpkt_body(qseq_ref, cseq_ref, table_ref, Ec_ref, h0_ref,
               W1a_ref, W1b_ref, b1_ref,
               W2a_ref, W2b_ref, W2c_ref, b2_ref,
               W3a_ref, W3b_ref, W3c_ref, b3_ref,
               W4a_ref, W4b_ref, b4_ref,
               wab_ref, bab_ref, Wdiff_ref, bdiff_ref,
               wdisc_ref, bdisc_ref,
               out_ref,
               q_all, learn_all, diff_all, disc_all, h_scr, al_all):
    f32 = jnp.float32
    _TC = 4
    _R = _BBLK * _TC
    iota_nq = jax.lax.broadcasted_iota(jnp.int32, (_NQ, _R), 0)
    iota_2 = jax.lax.broadcasted_iota(jnp.int32, (2, _R), 0)
    lane_iota = jax.lax.broadcasted_iota(jnp.int32, (1, _S), 1)

    def pre_body(r, _):
        idx = qseq_ref[pl.ds(0, 1), pl.ds(r, 1), :][0]
        ohT = (iota_nq == idx).astype(f32)
        ge = jax.lax.dot_general(ohT, table_ref[...],
                                 (((0,), (0,)), ((), ())),
                                 preferred_element_type=f32)
        q4 = ge[:, :_C]
        e4 = ge[:, _C:]
        cidx = cseq_ref[pl.ds(0, 1), pl.ds(r, 1), :][0]
        ohc = (iota_2 == cidx).astype(f32)
        c4 = jax.lax.dot_general(ohc, Ec_ref[...],
                                 (((0,), (0,)), ((), ())),
                                 preferred_element_type=f32)
        learning4 = (jnp.dot(e4, W1a_ref[...], preferred_element_type=f32)
                     + jnp.dot(c4, W1b_ref[...], preferred_element_type=f32)
                     + b1_ref[...])
        diff4 = jax.nn.sigmoid(
            jnp.dot(e4, Wdiff_ref[...], preferred_element_type=f32)
            + bdiff_ref[...])
        disc4 = jax.nn.sigmoid(
            jnp.dot(e4, wdisc_ref[...], preferred_element_type=f32)
            + bdisc_ref[...]) * 5.0
        t0 = r * _TC
        q_all[pl.ds(t0, _TC)] = q4.reshape(_TC, _BBLK, _C)
        learn_all[pl.ds(t0, _TC)] = learning4.reshape(_TC, _BBLK, _K)
        diff_all[pl.ds(t0, _TC)] = diff4.reshape(_TC, _BBLK, _C)
        disc_all[pl.ds(t0, _TC)] = disc4.reshape(_TC, _BBLK, _C)
        return 0

    jax.lax.fori_loop(0, _S // _TC, pre_body, 0, unroll=2)

    h0 = h0_ref[...]
    h_scr[...] = jnp.broadcast_to(h0[None], (_BBLK, _C, _K))
    q0 = q_all[pl.ds(0, 1)][0]
    h_tilde0 = jnp.dot(q0, h0, preferred_element_type=f32)
    wab3 = wab_ref[...][None]
    al_all[pl.ds(0, 1)] = jnp.zeros((1, _BBLK, _C), f32)
    bdiag = (jax.lax.broadcasted_iota(jnp.int32, (_BBLK, _BBLK * _C), 1)
             // _C == jax.lax.broadcasted_iota(
                 jnp.int32, (_BBLK, _BBLK * _C), 0)).astype(f32)

    def step(t, carry):
        h_tilde_pre, learning_pre, learning, q_e = carry
        q_next = q_all[pl.ds(t, 1)][0]
        lg_lin = (jnp.dot(learning_pre, W2a_ref[...], preferred_element_type=f32)
                  + jnp.dot(learning, W2b_ref[...], preferred_element_type=f32)
                  + jnp.dot(h_tilde_pre, W2c_ref[...], preferred_element_type=f32)
                  + b2_ref[...])
        gl_lin = (jnp.dot(learning_pre, W3a_ref[...], preferred_element_type=f32)
                  + jnp.dot(learning, W3b_ref[...], preferred_element_type=f32)
                  + jnp.dot(h_tilde_pre, W3c_ref[...], preferred_element_type=f32)
                  + b3_ref[...])
        LG = jax.nn.sigmoid(gl_lin) * (jnp.tanh(lg_lin) + 1.0) * 0.5
        h_pre = h_scr[...]
        hp2 = h_pre.reshape(_BBLK * _C, _K)
        gf_lin = (jnp.dot(hp2, W4a_ref[...],
                          preferred_element_type=f32).reshape(_BBLK, _C, _K)
                  + (jnp.dot(LG, W4b_ref[...], preferred_element_type=f32)
                     + b4_ref[...])[:, None, :])
        h = (q_e[:, :, None] * LG[:, None, :]
             + jax.nn.sigmoid(gf_lin) * h_pre)
        h_scr[...] = h
        al_all[pl.ds(t, 1)] = jnp.sum(h * wab3, axis=2)[None]
        sel = bdiag * jnp.tile(q_next, (1, _BBLK))
        h_tilde = jnp.dot(sel, h.reshape(_BBLK * _C, _K),
                          preferred_element_type=f32)
        learning_next = learn_all[pl.ds(t, 1)][0]
        return (h_tilde, learning, learning_next, q_next)

    init = (h_tilde0, jnp.zeros((_BBLK, _K), f32),
            learn_all[pl.ds(0, 1)][0], q0)
    jax.lax.fori_loop(1, _S, step, init, unroll=6)

    ab = jax.nn.sigmoid(al_all[...] + bab_ref[...])
    q3 = q_all[...]
    contrib = disc_all[...] * (ab - diff_all[...]) * q3
    yl = jnp.sum(contrib, axis=2)
    y = jax.nn.sigmoid(yl)
    out_ref[...] = jnp.where(lane_iota == 0, 0.0, y.T)


def kernel(question_seq, correctness_seq, q_matrix, E_q, E_c, h0,
           W1, b1, W2, b2, W3, b3, W4, b4,
           Wab, bab, Wdiff, bdiff, Wdisc, bdisc):
    f32 = jnp.float32
    qseq_r = (question_seq.T.reshape(_S // 4, 4, _NPROG, _BBLK)
              .transpose(2, 0, 1, 3).reshape(_NPROG, _S // 4, 4 * _BBLK))
    cseq_r = (correctness_seq.T.reshape(_S // 4, 4, _NPROG, _BBLK)
              .transpose(2, 0, 1, 3).reshape(_NPROG, _S // 4, 4 * _BBLK))
    table = jnp.concatenate([q_matrix, E_q], axis=1)

    full = lambda shape: pl.BlockSpec(shape, lambda i: (0,) * len(shape))
    grid_spec = pltpu.PrefetchScalarGridSpec(
        num_scalar_prefetch=0,
        grid=(_NPROG,),
        in_specs=[
            pl.BlockSpec((1, _S // 4, 4 * _BBLK), lambda i: (i, 0, 0)),
            pl.BlockSpec((1, _S // 4, 4 * _BBLK), lambda i: (i, 0, 0)),
            full((_NQ, _C + _DE)),
            full((2, _DC)),
            full((_C, _K)),
            full((_DE, _K)), full((_DC, _K)), full((1, _K)),
            full((_K, _K)), full((_K, _K)), full((_K, _K)), full((1, _K)),
            full((_K, _K)), full((_K, _K)), full((_K, _K)), full((1, _K)),
            full((_K, _K)), full((_K, _K)), full((1, _K)),
            full((1, _K)), full((1, 1)),
            full((_DE, _C)), full((1, _C)),
            full((_DE, _C)), full((1, 1)),
        ],
        out_specs=pl.BlockSpec((_BBLK, _S), lambda i: (i, 0)),
        scratch_shapes=[
            pltpu.VMEM((_S, _BBLK, _C), f32),
            pltpu.VMEM((_S, _BBLK, _K), f32),
            pltpu.VMEM((_S, _BBLK, _C), f32),
            pltpu.VMEM((_S, _BBLK, _C), f32),
            pltpu.VMEM((_BBLK, _C, _K), f32),
            pltpu.VMEM((_S, _BBLK, _C), f32),
        ],
    )
    pred = pl.pallas_call(
        _lpkt_body,
        grid_spec=grid_spec,
        out_shape=jax.ShapeDtypeStruct((_B, _S), f32),
        compiler_params=pltpu.CompilerParams(
            dimension_semantics=("parallel",),
            vmem_limit_bytes=48 * 1024 * 1024,
        ),
    )(
        qseq_r, cseq_r, table, E_c, h0,
        W1[:_DE], W1[_DE:], b1.reshape(1, _K),
        W2[:_K], W2[_K:2 * _K], W2[2 * _K:], b2.reshape(1, _K),
        W3[:_K], W3[_K:2 * _K], W3[2 * _K:], b3.reshape(1, _K),
        W4[:_K], W4[_K:], b4.reshape(1, _K),
        Wab.reshape(1, _K), bab.reshape(1, 1),
        Wdiff, bdiff.reshape(1, _C),
        jnp.tile(Wdisc, (1, _C)), bdisc.reshape(1, 1),
    )
    return pred
